# baseline (speedup 1.0000x reference)
import jax
import jax.numpy as jnp
from jax import lax
from jax.experimental import pallas as pl
from jax.experimental.pallas import tpu as pltpu
from jax.experimental.pallas import tpu_sc as plsc

VOCAB = 1000000
MAXLEN = 200
DIM = 64
BATCH = 4096

NUM_CORES = 2
NUM_SUBCORES = 16
NUM_WORKERS = NUM_CORES * NUM_SUBCORES
LANES = 16
BBLK = 128
NBT = BATCH // BBLK
NUM_BLOCKS = MAXLEN * NBT
BLOCKS_PER_WORKER = NUM_BLOCKS // NUM_WORKERS
NGROUPS = BBLK // LANES


def _load_indices(x_t_hbm, l, b0, idx_b, pidx_b):
    pltpu.sync_copy(x_t_hbm.at[l, pl.ds(b0, BBLK)], idx_b)
    for g in range(NGROUPS):
        sl = pl.ds(g * LANES, LANES)
        pidx_b[sl] = lax.shift_right_logical(idx_b[sl], 1)


def _compute_tile(rows_b, idx_b, pos_v, out_b, l):
    iotas = [lax.iota(jnp.int32, LANES) + g * LANES for g in range(NGROUPS)]
    halves = [
        lax.shift_left((idx_b[pl.ds(g * LANES, LANES)] & 1), 6)
        for g in range(NGROUPS)
    ]
    lsplat = jnp.full((LANES,), l, jnp.int32)

    @pl.loop(0, DIM)
    def _(c):
        csplat = jnp.full((LANES,), c, jnp.int32)
        pos_c = plsc.load_gather(pos_v, [csplat, lsplat])
        for g in range(NGROUPS):
            vals = plsc.load_gather(rows_b, [iotas[g], halves[g] + csplat])
            out_b[c, pl.ds(g * LANES, LANES)] = vals + pos_c


def _body(x_t_hbm, tpairs_hbm, pos_t_hbm, out_t_hbm,
          pos_v, idx_v, pidx_v, rows_v, out_v, gsem, osem):
    wid = lax.axis_index("s") * NUM_CORES + lax.axis_index("c")
    base = wid * BLOCKS_PER_WORKER

    pltpu.sync_copy(pos_t_hbm, pos_v)

    def coords(blk):
        return blk // NBT, (blk % NBT) * BBLK

    l0, b00 = coords(base)
    _load_indices(x_t_hbm, l0, b00, idx_v[0], pidx_v[0])
    pltpu.async_copy(tpairs_hbm.at[pidx_v[0]], rows_v[0], gsem[0])

    @pl.loop(0, BLOCKS_PER_WORKER, step=2)
    def _(t0):
        for p in range(2):
            q = 1 - p
            blk = base + t0 + p
            l, b0 = coords(blk)

            @pl.when(t0 + p + 1 < BLOCKS_PER_WORKER)
            def _():
                ln, b0n = coords(blk + 1)
                _load_indices(x_t_hbm, ln, b0n, idx_v[q], pidx_v[q])
                pltpu.async_copy(tpairs_hbm.at[pidx_v[q]], rows_v[q], gsem[q])

            pltpu.make_async_copy(tpairs_hbm.at[pidx_v[p]], rows_v[p],
                                  gsem[p]).wait()

            @pl.when(t0 + p >= 2)
            def _():
                lo, b0o = coords(blk - 2)
                pltpu.make_async_copy(
                    out_v[p], out_t_hbm.at[lo, :, pl.ds(b0o, BBLK)],
                    osem[p]).wait()

            _compute_tile(rows_v[p], idx_v[p], pos_v, out_v[p], l)
            pltpu.async_copy(out_v[p], out_t_hbm.at[l, :, pl.ds(b0, BBLK)],
                             osem[p])

    for p in range(2):
        ld, b0d = coords(base + BLOCKS_PER_WORKER - 2 + p)
        pltpu.make_async_copy(out_v[p], out_t_hbm.at[ld, :, pl.ds(b0d, BBLK)],
                              osem[p]).wait()


@jax.jit
def _sc_embed(x, token_table, pos_table):
    x_t = x.T
    tpairs = token_table.reshape(VOCAB // 2, 2 * DIM)
    pos_t = pos_table.T

    mesh = plsc.VectorSubcoreMesh(core_axis_name="c", subcore_axis_name="s")
    run = pl.kernel(
        _body,
        out_type=jax.ShapeDtypeStruct((MAXLEN, DIM, BATCH), jnp.float32),
        mesh=mesh,
        compiler_params=pltpu.CompilerParams(use_tc_tiling_on_sc=True,
                                             needs_layout_passes=False),
        scratch_types=[
            pltpu.VMEM((DIM, MAXLEN), jnp.float32),
            [pltpu.VMEM((BBLK,), jnp.int32)] * 2,
            [pltpu.VMEM((BBLK,), jnp.int32)] * 2,
            [pltpu.VMEM((BBLK, 2 * DIM), jnp.float32)] * 2,
            [pltpu.VMEM((DIM, BBLK), jnp.float32)] * 2,
            [pltpu.SemaphoreType.DMA] * 2,
            [pltpu.SemaphoreType.DMA] * 2,
        ],
    )
    out_t = run(x_t, tpairs, pos_t)
    return jnp.transpose(out_t, (2, 0, 1))


def kernel(x, token_table, pos_table):
    return _sc_embed(x.astype(jnp.int32), token_table, pos_table)


# speedup vs baseline: 1.4668x; 1.4668x over previous
import jax
import jax.numpy as jnp
from jax import lax
from jax.experimental import pallas as pl
from jax.experimental.pallas import tpu as pltpu
from jax.experimental.pallas import tpu_sc as plsc

VOCAB = 1000000
MAXLEN = 200
DIM = 64
BATCH = 4096

NUM_CORES = 2
NUM_SUBCORES = 16
NUM_WORKERS = NUM_CORES * NUM_SUBCORES
LANES = 16
BBLK = 128
NBT = BATCH // BBLK
NUM_BLOCKS = MAXLEN * NBT
BLOCKS_PER_WORKER = NUM_BLOCKS // NUM_WORKERS
NGROUPS = BBLK // LANES


def _load_indices(x_t_hbm, l, b0, idx_b, pidx_b):
    pltpu.sync_copy(x_t_hbm.at[l, pl.ds(b0, BBLK)], idx_b)
    for g in range(NGROUPS):
        sl = pl.ds(g * LANES, LANES)
        pidx_b[sl] = lax.shift_right_logical(idx_b[sl], 1)


def _compute_tile(rows_b, idx_b, pos_v, out_b, l):
    iotas = [lax.iota(jnp.int32, LANES) + g * LANES for g in range(NGROUPS)]
    halves = [
        lax.shift_left((idx_b[pl.ds(g * LANES, LANES)] & 1), 6)
        for g in range(NGROUPS)
    ]
    lsplat = jnp.full((LANES,), l, jnp.int32)

    @plsc.parallel_loop(0, DIM, unroll=4)
    def _(c):
        csplat = jnp.full((LANES,), c, jnp.int32)
        pos_c = plsc.load_gather(pos_v, [csplat, lsplat])
        for g in range(NGROUPS):
            vals = plsc.load_gather(rows_b, [iotas[g], halves[g] + csplat])
            out_b[c, pl.ds(g * LANES, LANES)] = vals + pos_c


def _body(x_t_hbm, tpairs_hbm, pos_t_hbm, out_t_hbm,
          pos_v, idx_v, pidx_v, rows_v, out_v, gsem, osem):
    wid = lax.axis_index("s") * NUM_CORES + lax.axis_index("c")
    base = wid * BLOCKS_PER_WORKER

    pltpu.sync_copy(pos_t_hbm, pos_v)

    def coords(blk):
        return blk // NBT, (blk % NBT) * BBLK

    l0, b00 = coords(base)
    _load_indices(x_t_hbm, l0, b00, idx_v[0], pidx_v[0])
    pltpu.async_copy(tpairs_hbm.at[pidx_v[0]], rows_v[0], gsem[0])

    @pl.loop(0, BLOCKS_PER_WORKER, step=2)
    def _(t0):
        for p in range(2):
            q = 1 - p
            blk = base + t0 + p
            l, b0 = coords(blk)

            @pl.when(t0 + p + 1 < BLOCKS_PER_WORKER)
            def _():
                ln, b0n = coords(blk + 1)
                _load_indices(x_t_hbm, ln, b0n, idx_v[q], pidx_v[q])
                pltpu.async_copy(tpairs_hbm.at[pidx_v[q]], rows_v[q], gsem[q])

            pltpu.make_async_copy(tpairs_hbm.at[pidx_v[p]], rows_v[p],
                                  gsem[p]).wait()

            @pl.when(t0 + p >= 2)
            def _():
                lo, b0o = coords(blk - 2)
                pltpu.make_async_copy(
                    out_v[p], out_t_hbm.at[lo, :, pl.ds(b0o, BBLK)],
                    osem[p]).wait()

            _compute_tile(rows_v[p], idx_v[p], pos_v, out_v[p], l)
            pltpu.async_copy(out_v[p], out_t_hbm.at[l, :, pl.ds(b0, BBLK)],
                             osem[p])

    for p in range(2):
        ld, b0d = coords(base + BLOCKS_PER_WORKER - 2 + p)
        pltpu.make_async_copy(out_v[p], out_t_hbm.at[ld, :, pl.ds(b0d, BBLK)],
                              osem[p]).wait()


@jax.jit
def _sc_embed(x, token_table, pos_table):
    x_t = x.T
    tpairs = token_table.reshape(VOCAB // 2, 2 * DIM)
    pos_t = pos_table.T

    mesh = plsc.VectorSubcoreMesh(core_axis_name="c", subcore_axis_name="s")
    run = pl.kernel(
        _body,
        out_type=jax.ShapeDtypeStruct((MAXLEN, DIM, BATCH), jnp.float32),
        mesh=mesh,
        compiler_params=pltpu.CompilerParams(use_tc_tiling_on_sc=True,
                                             needs_layout_passes=False),
        scratch_types=[
            pltpu.VMEM((DIM, MAXLEN), jnp.float32),
            [pltpu.VMEM((BBLK,), jnp.int32)] * 2,
            [pltpu.VMEM((BBLK,), jnp.int32)] * 2,
            [pltpu.VMEM((BBLK, 2 * DIM), jnp.float32)] * 2,
            [pltpu.VMEM((DIM, BBLK), jnp.float32)] * 2,
            [pltpu.SemaphoreType.DMA] * 2,
            [pltpu.SemaphoreType.DMA] * 2,
        ],
    )
    out_t = run(x_t, tpairs, pos_t)
    return jnp.transpose(out_t, (2, 0, 1))


def kernel(x, token_table, pos_table):
    return _sc_embed(x.astype(jnp.int32), token_table, pos_table)


# speedup vs baseline: 1.4811x; 1.0098x over previous
import jax
import jax.numpy as jnp
from jax import lax
from jax.experimental import pallas as pl
from jax.experimental.pallas import tpu as pltpu
from jax.experimental.pallas import tpu_sc as plsc

VOCAB = 1000000
MAXLEN = 200
DIM = 64
BATCH = 4096

NUM_CORES = 2
NUM_SUBCORES = 16
NUM_WORKERS = NUM_CORES * NUM_SUBCORES
LANES = 16
BBLK = 128
NBT = BATCH // BBLK
NUM_BLOCKS = MAXLEN * NBT
BLOCKS_PER_WORKER = NUM_BLOCKS // NUM_WORKERS
NGROUPS = BBLK // LANES
SUBL = 8


def _compute_tile(rows_b, pos_b, out_b, l):
    iotas = [lax.iota(jnp.int32, LANES) + g * LANES for g in range(NGROUPS)]
    lsplat = jnp.full((LANES,), l, jnp.int32)

    @plsc.parallel_loop(0, DIM, unroll=4)
    def _(c):
        csplat = jnp.full((LANES,), c, jnp.int32)
        pos_c = plsc.load_gather(pos_b, [lsplat, csplat])
        ct = c // SUBL
        ci = c % SUBL
        for g in range(NGROUPS):
            vals = plsc.load_gather(rows_b, [iotas[g], csplat])
            out_b[ct, ci, pl.ds(g * LANES, LANES)] = vals + pos_c


def _body(x5_hbm, tok_hbm, pos_hbm, out5_hbm,
          pos_v, idx_v, rows_v, out_v, gsem, osem):
    wid = lax.axis_index("s") * NUM_CORES + lax.axis_index("c")
    base = wid * BLOCKS_PER_WORKER

    pltpu.sync_copy(pos_hbm, pos_v)

    def coords(blk):
        l = blk // NBT
        return l, blk % NBT

    def load_indices(blk, p):
        l, bt = coords(blk)
        pltpu.sync_copy(x5_hbm.at[l // SUBL, bt, l % SUBL], idx_v[p])

    def start_gather(p):
        pltpu.async_copy(tok_hbm.at[idx_v[p]], rows_v[p], gsem[p])

    def wait_gather(p):
        pltpu.make_async_copy(tok_hbm.at[idx_v[p]], rows_v[p], gsem[p]).wait()

    def out_copy(p, blk):
        l, bt = coords(blk)
        return pltpu.make_async_copy(out_v[p], out5_hbm.at[l, :, bt], osem[p])

    load_indices(base, 0)
    start_gather(0)

    @pl.loop(0, BLOCKS_PER_WORKER, step=2)
    def _(t0):
        for p in range(2):
            q = 1 - p
            blk = base + t0 + p
            l, _ = coords(blk)

            @pl.when(t0 + p + 1 < BLOCKS_PER_WORKER)
            def _():
                load_indices(blk + 1, q)
                start_gather(q)

            wait_gather(p)

            @pl.when(t0 + p >= 2)
            def _():
                out_copy(p, blk - 2).wait()

            _compute_tile(rows_v[p], pos_v, out_v[p], l)
            out_copy(p, blk).start()

    for p in range(2):
        out_copy(p, base + BLOCKS_PER_WORKER - 2 + p).wait()


@jax.jit
def _sc_embed(x, token_table, pos_table):
    x5 = x.T.reshape(MAXLEN // SUBL, SUBL, NBT, BBLK).transpose(0, 2, 1, 3)

    mesh = plsc.VectorSubcoreMesh(core_axis_name="c", subcore_axis_name="s")
    run = pl.kernel(
        _body,
        out_type=jax.ShapeDtypeStruct(
            (MAXLEN, DIM // SUBL, NBT, SUBL, BBLK), jnp.float32),
        mesh=mesh,
        compiler_params=pltpu.CompilerParams(use_tc_tiling_on_sc=False,
                                             needs_layout_passes=False),
        scratch_types=[
            pltpu.VMEM((MAXLEN, DIM), jnp.float32),
            [pltpu.VMEM((BBLK,), jnp.int32)] * 2,
            [pltpu.VMEM((BBLK, DIM), jnp.float32)] * 2,
            [pltpu.VMEM((DIM // SUBL, SUBL, BBLK), jnp.float32)] * 2,
            [pltpu.SemaphoreType.DMA] * 2,
            [pltpu.SemaphoreType.DMA] * 2,
        ],
    )
    out5 = run(x5, token_table, pos_table)
    return out5.transpose(2, 4, 0, 1, 3).reshape(BATCH, MAXLEN, DIM)


def kernel(x, token_table, pos_table):
    return _sc_embed(x.astype(jnp.int32), token_table, pos_table)


# speedup vs baseline: 2.1881x; 1.4773x over previous
import jax
import jax.numpy as jnp
from jax import lax
from jax.experimental import pallas as pl
from jax.experimental.pallas import tpu as pltpu
from jax.experimental.pallas import tpu_sc as plsc

VOCAB = 1000000
MAXLEN = 200
DIM = 64
BATCH = 4096

NUM_CORES = 2
NUM_SUBCORES = 16
NUM_WORKERS = NUM_CORES * NUM_SUBCORES
LANES = 16
BBLK = 128
NBT = BATCH // BBLK
NUM_BLOCKS = MAXLEN * NBT
BLOCKS_PER_WORKER = NUM_BLOCKS // NUM_WORKERS
NGROUPS = BBLK // LANES
SUBL = 8


def _compute_tile(rows_b, pos_b, out_b, l):
    iota = lax.iota(jnp.int32, LANES)
    row_groups = [iota + g * LANES for g in range(NGROUPS)]
    lsplat = jnp.full((LANES,), l, jnp.int32)

    @plsc.parallel_loop(0, LANES, unroll=2)
    def _(d):
        coff = (iota + d) & (LANES - 1)
        for k in range(DIM // LANES):
            cvec = coff + (k * LANES)
            ct = lax.shift_right_logical(cvec, 3)
            ci = cvec & (SUBL - 1)
            pos_dk = plsc.load_gather(pos_b, [lsplat, cvec])
            for g in range(NGROUPS):
                vals = plsc.load_gather(rows_b, [row_groups[g], cvec])
                plsc.store_scatter(out_b, [ct, ci, row_groups[g]],
                                   vals + pos_dk)


def _body(x5_hbm, tok_hbm, pos_hbm, out5_hbm,
          pos_v, idx_v, rows_v, out_v, gsem, osem):
    wid = lax.axis_index("s") * NUM_CORES + lax.axis_index("c")
    base = wid * BLOCKS_PER_WORKER

    pltpu.sync_copy(pos_hbm, pos_v)

    def coords(blk):
        l = blk // NBT
        return l, blk % NBT

    def load_indices(blk, p):
        l, bt = coords(blk)
        pltpu.sync_copy(x5_hbm.at[l // SUBL, bt, l % SUBL], idx_v[p])

    def start_gather(p):
        pltpu.async_copy(tok_hbm.at[idx_v[p]], rows_v[p], gsem[p])

    def wait_gather(p):
        pltpu.make_async_copy(tok_hbm.at[idx_v[p]], rows_v[p], gsem[p]).wait()

    def out_copy(p, blk):
        l, bt = coords(blk)
        return pltpu.make_async_copy(out_v[p], out5_hbm.at[l, :, bt], osem[p])

    load_indices(base, 0)
    start_gather(0)

    @pl.loop(0, BLOCKS_PER_WORKER, step=2)
    def _(t0):
        for p in range(2):
            q = 1 - p
            blk = base + t0 + p
            l, _ = coords(blk)

            @pl.when(t0 + p + 1 < BLOCKS_PER_WORKER)
            def _():
                load_indices(blk + 1, q)
                start_gather(q)

            wait_gather(p)

            @pl.when(t0 + p >= 2)
            def _():
                out_copy(p, blk - 2).wait()

            _compute_tile(rows_v[p], pos_v, out_v[p], l)
            out_copy(p, blk).start()

    for p in range(2):
        out_copy(p, base + BLOCKS_PER_WORKER - 2 + p).wait()


@jax.jit
def _sc_embed(x, token_table, pos_table):
    x5 = x.T.reshape(MAXLEN // SUBL, SUBL, NBT, BBLK).transpose(0, 2, 1, 3)

    mesh = plsc.VectorSubcoreMesh(core_axis_name="c", subcore_axis_name="s")
    run = pl.kernel(
        _body,
        out_type=jax.ShapeDtypeStruct(
            (MAXLEN, DIM // SUBL, NBT, SUBL, BBLK), jnp.float32),
        mesh=mesh,
        compiler_params=pltpu.CompilerParams(use_tc_tiling_on_sc=False,
                                             needs_layout_passes=False),
        scratch_types=[
            pltpu.VMEM((MAXLEN, DIM), jnp.float32),
            [pltpu.VMEM((BBLK,), jnp.int32)] * 2,
            [pltpu.VMEM((BBLK, DIM), jnp.float32)] * 2,
            [pltpu.VMEM((DIM // SUBL, SUBL, BBLK), jnp.float32)] * 2,
            [pltpu.SemaphoreType.DMA] * 2,
            [pltpu.SemaphoreType.DMA] * 2,
        ],
    )
    out5 = run(x5, token_table, pos_table)
    return out5.transpose(2, 4, 0, 1, 3).reshape(BATCH, MAXLEN, DIM)


def kernel(x, token_table, pos_table):
    return _sc_embed(x.astype(jnp.int32), token_table, pos_table)


# speedup vs baseline: 2.3188x; 1.0597x over previous
import jax
import jax.numpy as jnp
from jax import lax
from jax.experimental import pallas as pl
from jax.experimental.pallas import tpu as pltpu
from jax.experimental.pallas import tpu_sc as plsc

VOCAB = 1000000
MAXLEN = 200
DIM = 64
BATCH = 4096

NUM_CORES = 2
NUM_SUBCORES = 16
NUM_WORKERS = NUM_CORES * NUM_SUBCORES
LANES = 16
BBLK = 128
NBT = BATCH // BBLK
NUM_BLOCKS = MAXLEN * NBT
BLOCKS_PER_WORKER = NUM_BLOCKS // NUM_WORKERS
NGROUPS = BBLK // LANES
SUBL = 8


def _compute_tile(rows_b, pos_b, out_b, l):
    iota = lax.iota(jnp.int32, LANES)
    row_groups = [iota + g * LANES for g in range(NGROUPS)]
    lsplat = jnp.full((LANES,), l, jnp.int32)

    @plsc.parallel_loop(0, LANES, unroll=2)
    def _(d):
        coff = (iota + d) & (LANES - 1)
        for k in range(DIM // LANES):
            cvec = coff + (k * LANES)
            ct = lax.shift_right_logical(cvec, 3)
            ci = cvec & (SUBL - 1)
            pos_dk = plsc.load_gather(pos_b, [lsplat, cvec])
            for g in range(NGROUPS):
                vals = plsc.load_gather(rows_b, [row_groups[g], cvec])
                plsc.store_scatter(out_b, [ct, ci, row_groups[g]],
                                   vals + pos_dk)


def _body(x5_hbm, tok_hbm, pos_hbm, out5_hbm,
          pos_v, idx_v, rows_v, out_v, gsem, osem):
    wid = lax.axis_index("s") * NUM_CORES + lax.axis_index("c")
    base = wid * BLOCKS_PER_WORKER

    pltpu.sync_copy(pos_hbm, pos_v)

    def coords(blk):
        l = blk // NBT
        return l, blk % NBT

    def load_indices(blk, p):
        l, bt = coords(blk)
        pltpu.sync_copy(x5_hbm.at[l // SUBL, bt, l % SUBL], idx_v[p])

    def start_gather(p):
        pltpu.async_copy(tok_hbm.at[idx_v[p]], rows_v[p], gsem[p])

    def wait_gather(p):
        pltpu.make_async_copy(tok_hbm.at[idx_v[p]], rows_v[p], gsem[p]).wait()

    def out_copy(p, blk):
        l, bt = coords(blk)
        return pltpu.make_async_copy(out_v[p], out5_hbm.at[l, :, bt], osem[p])

    load_indices(base, 0)
    start_gather(0)

    @pl.loop(0, BLOCKS_PER_WORKER, step=2)
    def _(t0):
        for p in range(2):
            q = 1 - p
            blk = base + t0 + p
            l, _ = coords(blk)

            @pl.when(t0 + p + 1 < BLOCKS_PER_WORKER)
            def _():
                load_indices(blk + 1, q)
                start_gather(q)

            wait_gather(p)

            @pl.when(t0 + p >= 2)
            def _():
                out_copy(p, blk - 2).wait()

            _compute_tile(rows_v[p], pos_v, out_v[p], l)
            out_copy(p, blk).start()

    for p in range(2):
        out_copy(p, base + BLOCKS_PER_WORKER - 2 + p).wait()


@jax.jit
def _sc_embed(x, token_table, pos_table):
    x5 = x.T.reshape(MAXLEN // SUBL, SUBL, NBT, BBLK).transpose(0, 2, 1, 3)
    tpad = jnp.pad(
        token_table.reshape(VOCAB // SUBL, SUBL, DIM),
        ((0, 0), (0, 0), (0, BBLK - DIM))).reshape(VOCAB, BBLK)

    mesh = plsc.VectorSubcoreMesh(core_axis_name="c", subcore_axis_name="s")
    run = pl.kernel(
        _body,
        out_type=jax.ShapeDtypeStruct(
            (MAXLEN, DIM // SUBL, NBT, SUBL, BBLK), jnp.float32),
        mesh=mesh,
        compiler_params=pltpu.CompilerParams(use_tc_tiling_on_sc=False,
                                             needs_layout_passes=False),
        scratch_types=[
            pltpu.VMEM((MAXLEN, DIM), jnp.float32),
            [pltpu.VMEM((BBLK,), jnp.int32)] * 2,
            [pltpu.VMEM((BBLK, BBLK), jnp.float32)] * 2,
            [pltpu.VMEM((DIM // SUBL, SUBL, BBLK), jnp.float32)] * 2,
            [pltpu.SemaphoreType.DMA] * 2,
            [pltpu.SemaphoreType.DMA] * 2,
        ],
    )
    out5 = run(x5, tpad, pos_table)
    return out5.transpose(2, 4, 0, 1, 3).reshape(BATCH, MAXLEN, DIM)


def kernel(x, token_table, pos_table):
    return _sc_embed(x.astype(jnp.int32), token_table, pos_table)


# speedup vs baseline: 3.4516x; 1.4886x over previous
import jax
import jax.numpy as jnp
from jax import lax
from jax.experimental import pallas as pl
from jax.experimental.pallas import tpu as pltpu
from jax.experimental.pallas import tpu_sc as plsc

VOCAB = 1000000
MAXLEN = 200
DIM = 64
BATCH = 4096

NUM_CORES = 2
NUM_SUBCORES = 16
NUM_WORKERS = NUM_CORES * NUM_SUBCORES
LANES = 16
BBLK = 128
NBT = BATCH // BBLK
NUM_BLOCKS = MAXLEN * NBT
BLOCKS_PER_WORKER = NUM_BLOCKS // NUM_WORKERS
NGROUPS = BBLK // LANES
SUBL = 8


def _compute_tile(rows_b, pos_b, out_b, l):
    iota = lax.iota(jnp.int32, LANES)
    row_groups = [iota + g * LANES for g in range(NGROUPS)]
    lsplat = jnp.full((LANES,), l, jnp.int32)

    @plsc.parallel_loop(0, LANES, unroll=2)
    def _(d):
        coff = (iota + d) & (LANES - 1)
        for k in range(DIM // LANES):
            cvec = coff + (k * LANES)
            ct = lax.shift_right_logical(cvec, 3)
            ci = cvec & (SUBL - 1)
            pos_dk = plsc.load_gather(pos_b, [lsplat, cvec])
            for g in range(NGROUPS):
                vals = plsc.load_gather(rows_b, [row_groups[g], cvec])
                plsc.store_scatter(out_b, [ct, ci, row_groups[g]],
                                   vals + pos_dk)


def _body(x5_hbm, tok_hbm, pos_hbm, out5_hbm,
          pos_v, idx_v, rows_v, out_v, gsem, osem):
    wid = lax.axis_index("s") * NUM_CORES + lax.axis_index("c")
    base = wid * BLOCKS_PER_WORKER

    pltpu.sync_copy(pos_hbm, pos_v)

    def coords(blk):
        l = blk // NBT
        return l, blk % NBT

    def load_indices(blk, p):
        l, bt = coords(blk)
        pltpu.sync_copy(x5_hbm.at[l // SUBL, bt, l % SUBL], idx_v[p])

    def start_gather(p):
        pltpu.async_copy(tok_hbm.at[idx_v[p]], rows_v[p], gsem[p])

    def wait_gather(p):
        pltpu.make_async_copy(tok_hbm.at[idx_v[p]], rows_v[p], gsem[p]).wait()

    def out_copy(p, blk):
        l, bt = coords(blk)
        return pltpu.make_async_copy(out_v[p], out5_hbm.at[l, :, bt], osem[p])

    load_indices(base, 0)
    start_gather(0)

    @pl.loop(0, BLOCKS_PER_WORKER, step=2)
    def _(t0):
        for p in range(2):
            q = 1 - p
            blk = base + t0 + p
            l, _ = coords(blk)

            @pl.when(t0 + p + 1 < BLOCKS_PER_WORKER)
            def _():
                load_indices(blk + 1, q)
                start_gather(q)

            wait_gather(p)

            @pl.when(t0 + p >= 2)
            def _():
                out_copy(p, blk - 2).wait()

            _compute_tile(rows_v[p], pos_v, out_v[p], l)
            out_copy(p, blk).start()

    for p in range(2):
        out_copy(p, base + BLOCKS_PER_WORKER - 2 + p).wait()


N_RT = VOCAB // BBLK
RT_PER_WORKER = N_RT // NUM_WORKERS
TAIL_TOK = VOCAB - N_RT * BBLK


def _relayout_tile(vin, vout, nrow_groups):
    iota = lax.iota(jnp.int32, LANES)
    half = iota >> 1
    parity64 = (iota & 1) * DIM

    @plsc.parallel_loop(0, LANES, unroll=2)
    def _(d):
        coff = (iota + d) & (LANES - 1)
        for k in range(DIM // LANES):
            cvec = coff + (k * LANES)
            dl = parity64 + cvec
            for g in range(nrow_groups):
                rvec = iota + g * LANES
                vals = plsc.load_gather(vin, [cvec, rvec])
                plsc.store_scatter(
                    vout, [jnp.full((LANES,), g, jnp.int32), half, dl], vals)


def _relayout_body(tt_hbm, o3_hbm, vin, vout, vin_t, isem, osem):
    wid = lax.axis_index("s") * NUM_CORES + lax.axis_index("c")

    def rt_of(i):
        return i * NUM_WORKERS + wid

    def in_copy(p, rt):
        return pltpu.make_async_copy(
            tt_hbm.at[:, pl.ds(rt * BBLK, BBLK)], vin[p], isem[p])

    def out_copy(p, rt):
        return pltpu.make_async_copy(vout[p], o3_hbm.at[pl.ds(rt * SUBL, SUBL)],
                                     osem[p])

    in_copy(0, rt_of(0)).start()

    @pl.loop(0, RT_PER_WORKER, step=2)
    def _(i0):
        for p in range(2):
            q = 1 - p
            i = i0 + p
            rt = rt_of(i)

            @pl.when(i + 1 < RT_PER_WORKER)
            def _():
                in_copy(q, rt_of(i + 1)).start()

            in_copy(p, rt).wait()

            @pl.when(i >= 2)
            def _():
                out_copy(p, rt_of(i - 2)).wait()

            _relayout_tile(vin[p], vout[p], NGROUPS)
            out_copy(p, rt).start()

    for p in range(2):
        out_copy(p, rt_of(RT_PER_WORKER - 2 + p)).wait()

    @pl.when(wid < N_RT - RT_PER_WORKER * NUM_WORKERS)
    def _():
        rt = RT_PER_WORKER * NUM_WORKERS + wid
        pltpu.sync_copy(tt_hbm.at[:, pl.ds(rt * BBLK, BBLK)], vin[0])
        _relayout_tile(vin[0], vout[0], NGROUPS)
        pltpu.sync_copy(vout[0], o3_hbm.at[pl.ds(rt * SUBL, SUBL)])

    @pl.when(wid == NUM_WORKERS - 1)
    def _():
        pltpu.sync_copy(tt_hbm.at[:, pl.ds(N_RT * BBLK, TAIL_TOK)], vin_t)
        _relayout_tile(vin_t, vout[1], TAIL_TOK // LANES)
        pltpu.sync_copy(vout[1].at[pl.ds(0, TAIL_TOK * DIM // (SUBL * BBLK))],
                        o3_hbm.at[pl.ds(N_RT * SUBL, TAIL_TOK * DIM
                                        // (SUBL * BBLK))])


def _sc_relayout(table_t):
    mesh = plsc.VectorSubcoreMesh(core_axis_name="c", subcore_axis_name="s")
    run = pl.kernel(
        _relayout_body,
        out_type=jax.ShapeDtypeStruct((VOCAB // LANES, SUBL, BBLK),
                                      jnp.float32),
        mesh=mesh,
        compiler_params=pltpu.CompilerParams(use_tc_tiling_on_sc=True,
                                             needs_layout_passes=False),
        scratch_types=[
            [pltpu.VMEM((DIM, BBLK), jnp.float32)] * 2,
            [pltpu.VMEM((SUBL, SUBL, BBLK), jnp.float32)] * 2,
            pltpu.VMEM((DIM, TAIL_TOK), jnp.float32),
            [pltpu.SemaphoreType.DMA] * 2,
            [pltpu.SemaphoreType.DMA] * 2,
        ],
    )
    return run(table_t).reshape(VOCAB, DIM)


@jax.jit
def _sc_embed(x, token_table, pos_table):
    x5 = x.T.reshape(MAXLEN // SUBL, SUBL, NBT, BBLK).transpose(0, 2, 1, 3)
    tpad = _sc_relayout(token_table.T)

    mesh = plsc.VectorSubcoreMesh(core_axis_name="c", subcore_axis_name="s")
    run = pl.kernel(
        _body,
        out_type=jax.ShapeDtypeStruct(
            (MAXLEN, DIM // SUBL, NBT, SUBL, BBLK), jnp.float32),
        mesh=mesh,
        compiler_params=pltpu.CompilerParams(use_tc_tiling_on_sc=False,
                                             needs_layout_passes=False),
        scratch_types=[
            pltpu.VMEM((MAXLEN, DIM), jnp.float32),
            [pltpu.VMEM((BBLK,), jnp.int32)] * 2,
            [pltpu.VMEM((BBLK, DIM), jnp.float32)] * 2,
            [pltpu.VMEM((DIM // SUBL, SUBL, BBLK), jnp.float32)] * 2,
            [pltpu.SemaphoreType.DMA] * 2,
            [pltpu.SemaphoreType.DMA] * 2,
        ],
    )
    out5 = run(x5, tpad, pos_table)
    return out5.transpose(2, 4, 0, 1, 3).reshape(BATCH, MAXLEN, DIM)


def kernel(x, token_table, pos_table):
    return _sc_embed(x.astype(jnp.int32), token_table, pos_table)
